# baseline (speedup 1.0000x reference)
import functools

import jax
import jax.numpy as jnp
from jax import lax
from jax.experimental import pallas as pl
from jax.experimental.pallas import tpu as pltpu
from jax.experimental.pallas import tpu_sc as plsc

EMBED = 64
B = 4096 * 50


@functools.lru_cache(maxsize=None)
def _build(chunk: int):
    info = plsc.get_sparse_core_info()
    nc, ns = info.num_cores, info.num_subcores
    nw = nc * ns
    b_per_w = B // nw
    n_chunks = b_per_w // chunk

    mesh = plsc.VectorSubcoreMesh(core_axis_name="c", subcore_axis_name="s")

    @functools.partial(
        pl.kernel,
        mesh=mesh,
        compiler_params=pltpu.CompilerParams(use_tc_tiling_on_sc=False),
        out_type=(
            jax.ShapeDtypeStruct((B, EMBED), jnp.float32),
            jax.ShapeDtypeStruct((B, EMBED), jnp.float32),
        ),
        scratch_types=[
            pltpu.VMEM((n_chunks, chunk), jnp.int32),
            pltpu.VMEM((chunk, EMBED), jnp.float32),
            pltpu.VMEM((chunk, EMBED), jnp.float32),
            pltpu.SemaphoreType.DMA,
            pltpu.SemaphoreType.DMA,
        ],
    )
    def gather_kernel(num_idx, unit_idx, num_tab, unit_tab,
                      num_out, unit_out, idx_v, rows0, rows1, sem0, sem1):
        wid = lax.axis_index("s") * nc + lax.axis_index("c")
        base = wid * b_per_w
        bufs = (rows0, rows1)
        sems = (sem0, sem1)
        for idx_hbm, tab_hbm, out_hbm in (
            (num_idx, num_tab, num_out),
            (unit_idx, unit_tab, unit_out),
        ):
            pltpu.sync_copy(idx_hbm.at[wid], idx_v)
            pltpu.async_copy(tab_hbm.at[idx_v.at[0]], bufs[0], sems[0])

            def body(g, _, tab_hbm=tab_hbm, out_hbm=out_hbm):
                for b in range(2):
                    c = g * 2 + b
                    nb = (b + 1) % 2
                    pltpu.make_async_copy(
                        tab_hbm.at[idx_v.at[c]], bufs[b], sems[b]).wait()

                    @pl.when(c + 1 < n_chunks)
                    def _(nb=nb, c=c, tab_hbm=tab_hbm):
                        pltpu.async_copy(
                            tab_hbm.at[idx_v.at[c + 1]], bufs[nb], sems[nb])

                    pltpu.sync_copy(
                        bufs[b], out_hbm.at[pl.ds(base + c * chunk, chunk)])
                return 0

            lax.fori_loop(0, n_chunks // 2, body, 0)

    return gather_kernel, nw, n_chunks


def kernel(num_tokens, unit_tokens, num_table, unit_table):
    chunk = 800
    fn, nw, n_chunks = _build(chunk)
    shape = num_tokens.shape
    num_idx = num_tokens.reshape(nw, n_chunks, chunk).astype(jnp.int32)
    unit_idx = unit_tokens.reshape(nw, n_chunks, chunk).astype(jnp.int32)
    num_out, unit_out = fn(num_idx, unit_idx, num_table, unit_table)
    return (num_out.reshape(*shape, EMBED), unit_out.reshape(*shape, EMBED))


# speedup vs baseline: 1.0651x; 1.0651x over previous
import functools

import jax
import jax.numpy as jnp
from jax import lax
from jax.experimental import pallas as pl
from jax.experimental.pallas import tpu as pltpu
from jax.experimental.pallas import tpu_sc as plsc

EMBED = 64
B = 4096 * 50


@functools.lru_cache(maxsize=None)
def _build(chunk: int):
    info = plsc.get_sparse_core_info()
    nc, ns = info.num_cores, info.num_subcores
    nw = nc * ns
    b_per_w = B // nw
    n_chunks = b_per_w // chunk

    mesh = plsc.VectorSubcoreMesh(core_axis_name="c", subcore_axis_name="s")

    @functools.partial(
        pl.kernel,
        mesh=mesh,
        compiler_params=pltpu.CompilerParams(use_tc_tiling_on_sc=False),
        out_type=jax.ShapeDtypeStruct((B, EMBED), jnp.float32),
        scratch_types=[
            pltpu.VMEM((n_chunks, chunk), jnp.int32),
            pltpu.VMEM((chunk, EMBED), jnp.float32),
            pltpu.VMEM((chunk, EMBED), jnp.float32),
            pltpu.SemaphoreType.DMA,
            pltpu.SemaphoreType.DMA,
        ],
    )
    def gather_one(idx_hbm, tab_hbm, out_hbm, idx_v, rows0, rows1, sem0, sem1):
        wid = lax.axis_index("s") * nc + lax.axis_index("c")
        base = wid * b_per_w
        bufs = (rows0, rows1)
        sems = (sem0, sem1)
        pltpu.sync_copy(idx_hbm.at[wid], idx_v)
        pltpu.async_copy(tab_hbm.at[idx_v.at[0]], bufs[0], sems[0])

        def body(g, _):
            for b in range(2):
                c = g * 2 + b
                nb = (b + 1) % 2
                pltpu.make_async_copy(
                    tab_hbm.at[idx_v.at[c]], bufs[b], sems[b]).wait()

                @pl.when(c + 1 < n_chunks)
                def _(nb=nb, c=c):
                    pltpu.async_copy(
                        tab_hbm.at[idx_v.at[c + 1]], bufs[nb], sems[nb])

                pltpu.sync_copy(
                    bufs[b], out_hbm.at[pl.ds(base + c * chunk, chunk)])
            return 0

        lax.fori_loop(0, n_chunks // 2, body, 0)

    return gather_one, nw, n_chunks


def kernel(num_tokens, unit_tokens, num_table, unit_table):
    chunk = 800
    fn, nw, n_chunks = _build(chunk)
    shape = num_tokens.shape
    scale = (num_tokens[0, 0] * 0 + 1).astype(jnp.float32)
    num_idx = num_tokens.reshape(nw, n_chunks, chunk).astype(jnp.int32)
    unit_idx = unit_tokens.reshape(nw, n_chunks, chunk).astype(jnp.int32)
    num_out = fn(num_idx, num_table)
    unit_out = fn(unit_idx, unit_table)
    return (num_out.reshape(*shape, EMBED) * scale,
            unit_out.reshape(*shape, EMBED) * scale)
